# initial kernel scaffold (unmeasured)
import jax
import jax.numpy as jnp
from jax import lax
from jax.experimental import pallas as pl
from jax.experimental.pallas import tpu as pltpu

N_DEV = 4
M_PER = 1024
K = 4096
N_PER = 2048


def kernel(x, w_mat):
    def body(x_hbm, w_ref, out_hbm, comm_hbm, x_vmem, out_tile,
             send_sems, recv_sems, copy_sem):
        my = lax.axis_index("i")
        left = lax.rem(my + N_DEV - 1, N_DEV)
        right = lax.rem(my + 1, N_DEV)

        barrier_sem = pltpu.get_barrier_semaphore()
        for nbr in (left, right):
            pl.semaphore_signal(
                barrier_sem, inc=1,
                device_id=(nbr,), device_id_type=pl.DeviceIdType.MESH,
            )
        pl.semaphore_wait(barrier_sem, 2)

        def gemm_chunk(src_hbm, origin):
            cp = pltpu.make_async_copy(src_hbm, x_vmem, copy_sem)
            cp.start()
            cp.wait()
            out_tile[...] = jnp.dot(
                x_vmem[...], w_ref[...], preferred_element_type=jnp.float32
            )
            cp = pltpu.make_async_copy(
                out_tile, out_hbm.at[pl.ds(origin * M_PER, M_PER), :], copy_sem
            )
            cp.start()
            cp.wait()

        gemm_chunk(x_hbm, my)

        for h in range(N_DEV - 1):
            o_send = lax.rem(my + N_DEV - h, N_DEV)
            o_recv = lax.rem(my + N_DEV - h - 1, N_DEV)
            src = x_hbm if h == 0 else comm_hbm.at[o_send]
            rdma = pltpu.make_async_remote_copy(
                src_ref=src,
                dst_ref=comm_hbm.at[o_send],
                send_sem=send_sems.at[h],
                recv_sem=recv_sems.at[h],
                device_id=(right,),
                device_id_type=pl.DeviceIdType.MESH,
            )
            rdma.start()
            rdma.wait()
            gemm_chunk(comm_hbm.at[o_recv], o_recv)

    return pl.pallas_call(
        body,
        out_shape=jax.ShapeDtypeStruct((N_DEV * M_PER, N_PER), jnp.float32),
        in_specs=[
            pl.BlockSpec(memory_space=pl.ANY),
            pl.BlockSpec(memory_space=pltpu.MemorySpace.VMEM),
        ],
        out_specs=pl.BlockSpec(memory_space=pl.ANY),
        scratch_shapes=[
            pltpu.MemorySpace.HBM((N_DEV, M_PER, K), jnp.float32),
            pltpu.MemorySpace.VMEM((M_PER, K), jnp.float32),
            pltpu.MemorySpace.VMEM((M_PER, N_PER), jnp.float32),
            pltpu.SemaphoreType.DMA((N_DEV - 1,)),
            pltpu.SemaphoreType.DMA((N_DEV - 1,)),
            pltpu.SemaphoreType.DMA,
        ],
        compiler_params=pltpu.CompilerParams(collective_id=0),
    )(x, w_mat)


# baseline (device time: 693195 ns/iter reference)
import jax
import jax.numpy as jnp
from jax import lax
from jax.experimental import pallas as pl
from jax.experimental.pallas import tpu as pltpu

N_DEV = 4
M_PER = 1024
K = 4096
N_PER = 2048


def kernel(x, w_mat):
    def body(x_hbm, w_ref, out_hbm, comm_hbm, x_vmem, out_tile,
             send_sems, recv_sems, copy_sem):
        my = lax.axis_index("i")
        left = lax.rem(my + N_DEV - 1, N_DEV)
        right = lax.rem(my + 1, N_DEV)

        barrier_sem = pltpu.get_barrier_semaphore()
        for nbr in (left, right):
            pl.semaphore_signal(
                barrier_sem, inc=1,
                device_id=(nbr,), device_id_type=pl.DeviceIdType.MESH,
            )
        pl.semaphore_wait(barrier_sem, 2)

        def gemm_chunk(src_hbm, origin):
            cp = pltpu.make_async_copy(src_hbm, x_vmem, copy_sem)
            cp.start()
            cp.wait()
            out_tile[...] = jnp.dot(
                x_vmem[...], w_ref[...], preferred_element_type=jnp.float32
            )
            cp = pltpu.make_async_copy(
                out_tile, out_hbm.at[pl.ds(origin * M_PER, M_PER), :], copy_sem
            )
            cp.start()
            cp.wait()

        gemm_chunk(x_hbm, my)

        for h in range(N_DEV - 1):
            o_send = lax.rem(my + N_DEV - h, N_DEV)
            o_recv = lax.rem(my + N_DEV - h - 1, N_DEV)
            src = x_hbm if h == 0 else comm_hbm.at[o_send]
            rdma = pltpu.make_async_remote_copy(
                src_ref=src,
                dst_ref=comm_hbm.at[o_send],
                send_sem=send_sems.at[h],
                recv_sem=recv_sems.at[h],
                device_id=(right,),
                device_id_type=pl.DeviceIdType.MESH,
            )
            rdma.start()
            rdma.wait()
            gemm_chunk(comm_hbm.at[o_recv], o_recv)

    out, _comm = pl.pallas_call(
        body,
        out_shape=(
            jax.ShapeDtypeStruct((N_DEV * M_PER, N_PER), jnp.float32),
            jax.ShapeDtypeStruct((N_DEV, M_PER, K), jnp.float32),
        ),
        in_specs=[
            pl.BlockSpec(memory_space=pl.ANY),
            pl.BlockSpec(memory_space=pltpu.MemorySpace.VMEM),
        ],
        out_specs=(
            pl.BlockSpec(memory_space=pl.ANY),
            pl.BlockSpec(memory_space=pl.ANY),
        ),
        scratch_shapes=[
            pltpu.MemorySpace.VMEM((M_PER, K), jnp.float32),
            pltpu.MemorySpace.VMEM((M_PER, N_PER), jnp.float32),
            pltpu.SemaphoreType.DMA((N_DEV - 1,)),
            pltpu.SemaphoreType.DMA((N_DEV - 1,)),
            pltpu.SemaphoreType.DMA,
        ],
        compiler_params=pltpu.CompilerParams(
            collective_id=0,
            vmem_limit_bytes=64 * 1024 * 1024,
        ),
    )(x, w_mat)
    return out


# device time: 338670 ns/iter; 2.0468x vs baseline; 2.0468x over previous
import jax
import jax.numpy as jnp
from jax import lax
from jax.experimental import pallas as pl
from jax.experimental.pallas import tpu as pltpu

N_DEV = 4
M_PER = 1024
H = 512
K = 4096
N_PER = 2048
N_HOP = N_DEV - 1


def kernel(x, w_mat):
    def body(x_hbm, w_ref, out_hbm, comm_hbm, xh_vmem, out_tile,
             sendR_sems, recvR_sems, sendL_sems, recvL_sems,
             in_sems, out_sems):
        my = lax.axis_index("i")
        left = lax.rem(my + N_DEV - 1, N_DEV)
        right = lax.rem(my + 1, N_DEV)

        barrier_sem = pltpu.get_barrier_semaphore()
        for nbr in (left, right):
            pl.semaphore_signal(
                barrier_sem, inc=1,
                device_id=(nbr,), device_id_type=pl.DeviceIdType.MESH,
            )
        pl.semaphore_wait(barrier_sem, 2)

        def top(ref, o):
            return ref.at[o, pl.ds(0, H), :]

        def bot(ref, o):
            return ref.at[o, pl.ds(H, H), :]

        send_descs = []

        def send_full(src, dst, ssem, rsem, to):
            rdma = pltpu.make_async_remote_copy(
                src_ref=src, dst_ref=dst, send_sem=ssem, recv_sem=rsem,
                device_id=(to,), device_id_type=pl.DeviceIdType.MESH,
            )
            rdma.start()
            send_descs.append(rdma)

        def wait_recv(dst, rsem):
            rdma = pltpu.make_async_remote_copy(
                src_ref=dst, dst_ref=dst, send_sem=rsem, recv_sem=rsem,
                device_id=(my,), device_id_type=pl.DeviceIdType.MESH,
            )
            rdma.wait_recv()

        pending_out = [None, None]
        state = {"t": 0}

        def gemm_half(src_hbm, row_start):
            t = state["t"]
            state["t"] = t + 1
            buf = t % 2
            cp_in = pltpu.make_async_copy(src_hbm, xh_vmem, in_sems)
            cp_in.start()
            if pending_out[buf] is not None:
                pending_out[buf].wait()
            cp_in.wait()
            out_tile[buf, :, :] = jnp.dot(
                xh_vmem[...], w_ref[...], preferred_element_type=jnp.float32
            )
            cp_out = pltpu.make_async_copy(
                out_tile.at[buf], out_hbm.at[pl.ds(row_start, H), :],
                out_sems.at[buf],
            )
            cp_out.start()
            pending_out[buf] = cp_out

        x_top = x_hbm.at[pl.ds(0, H), :]
        x_bot = x_hbm.at[pl.ds(H, H), :]
        send_full(x_top, top(comm_hbm, my), sendR_sems.at[0],
                  recvR_sems.at[0], right)
        send_full(x_bot, bot(comm_hbm, my), sendL_sems.at[0],
                  recvL_sems.at[0], left)

        gemm_half(x_top, my * M_PER)
        gemm_half(x_bot, my * M_PER + H)

        for h in range(N_HOP):
            o_r = lax.rem(my + N_DEV - h - 1, N_DEV)
            o_l = lax.rem(my + h + 1, N_DEV)
            wait_recv(top(comm_hbm, o_r), recvR_sems.at[h])
            if h + 1 < N_HOP:
                send_full(top(comm_hbm, o_r), top(comm_hbm, o_r),
                          sendR_sems.at[h + 1], recvR_sems.at[h + 1], right)
            wait_recv(bot(comm_hbm, o_l), recvL_sems.at[h])
            if h + 1 < N_HOP:
                send_full(bot(comm_hbm, o_l), bot(comm_hbm, o_l),
                          sendL_sems.at[h + 1], recvL_sems.at[h + 1], left)
            gemm_half(top(comm_hbm, o_r), o_r * M_PER)
            gemm_half(bot(comm_hbm, o_l), o_l * M_PER + H)

        for d in send_descs:
            d.wait_send()
        for cp in pending_out:
            if cp is not None:
                cp.wait()

    out, _comm = pl.pallas_call(
        body,
        out_shape=(
            jax.ShapeDtypeStruct((N_DEV * M_PER, N_PER), jnp.float32),
            jax.ShapeDtypeStruct((N_DEV, M_PER, K), jnp.float32),
        ),
        in_specs=[
            pl.BlockSpec(memory_space=pl.ANY),
            pl.BlockSpec(memory_space=pltpu.MemorySpace.VMEM),
        ],
        out_specs=(
            pl.BlockSpec(memory_space=pl.ANY),
            pl.BlockSpec(memory_space=pl.ANY),
        ),
        scratch_shapes=[
            pltpu.MemorySpace.VMEM((H, K), jnp.float32),
            pltpu.MemorySpace.VMEM((2, H, N_PER), jnp.float32),
            pltpu.SemaphoreType.DMA((N_HOP,)),
            pltpu.SemaphoreType.DMA((N_HOP,)),
            pltpu.SemaphoreType.DMA((N_HOP,)),
            pltpu.SemaphoreType.DMA((N_HOP,)),
            pltpu.SemaphoreType.DMA,
            pltpu.SemaphoreType.DMA((2,)),
        ],
        compiler_params=pltpu.CompilerParams(
            collective_id=0,
            vmem_limit_bytes=64 * 1024 * 1024,
        ),
    )(x, w_mat)
    return out


# device time: 317307 ns/iter; 2.1846x vs baseline; 1.0673x over previous
import jax
import jax.numpy as jnp
from jax import lax
from jax.experimental import pallas as pl
from jax.experimental.pallas import tpu as pltpu

N_DEV = 4
M_PER = 1024
H = 512
Q = 256
K = 4096
N_PER = 2048
N_HOP = N_DEV - 1
N_MSG = 4


def kernel(x, w_mat):
    def body(x_hbm, w_hbm, out_hbm, comm_hbm, w_vmem, xh_vmem, out_tile,
             sendR_sems, recvR_sems, sendL_sems, recvL_sems,
             w_sem, in_sem, out_sems):
        my = lax.axis_index("i")
        left = lax.rem(my + N_DEV - 1, N_DEV)
        right = lax.rem(my + 1, N_DEV)

        cp_w = pltpu.make_async_copy(w_hbm, w_vmem, w_sem)
        cp_w.start()

        barrier_sem = pltpu.get_barrier_semaphore()
        for nbr in (left, right):
            pl.semaphore_signal(
                barrier_sem, inc=1,
                device_id=(nbr,), device_id_type=pl.DeviceIdType.MESH,
            )
        pl.semaphore_wait(barrier_sem, 2)

        def top(o, off=0, rows=H):
            return comm_hbm.at[o, pl.ds(off, rows), :]

        def bot(o, off=0, rows=H):
            return comm_hbm.at[o, pl.ds(H + off, rows), :]

        send_descs = []

        def send(src, dst, ssem, rsem, to):
            rdma = pltpu.make_async_remote_copy(
                src_ref=src, dst_ref=dst, send_sem=ssem, recv_sem=rsem,
                device_id=(to,), device_id_type=pl.DeviceIdType.MESH,
            )
            rdma.start()
            send_descs.append(rdma)

        def wait_recv(dst, rsem):
            rdma = pltpu.make_async_remote_copy(
                src_ref=dst, dst_ref=dst, send_sem=rsem, recv_sem=rsem,
                device_id=(my,), device_id_type=pl.DeviceIdType.MESH,
            )
            rdma.wait_recv()

        pending_out = [None, None]
        state = {"t": 0}

        def gemm(src_hbm, row_start, rows=H):
            t = state["t"]
            state["t"] = t + 1
            buf = t % 2
            cp_in = pltpu.make_async_copy(
                src_hbm, xh_vmem.at[pl.ds(0, rows)], in_sem
            )
            cp_in.start()
            if pending_out[buf] is not None:
                pending_out[buf].wait()
            cp_in.wait()
            out_tile[buf, 0:rows, :] = jnp.dot(
                xh_vmem[0:rows], w_vmem[...],
                preferred_element_type=jnp.float32,
            )
            cp_out = pltpu.make_async_copy(
                out_tile.at[buf, pl.ds(0, rows)],
                out_hbm.at[pl.ds(row_start, rows), :],
                out_sems.at[buf],
            )
            cp_out.start()
            pending_out[buf] = cp_out

        send(x_hbm.at[pl.ds(0, H), :], top(my),
             sendR_sems.at[0], recvR_sems.at[0], right)
        send(x_hbm.at[pl.ds(H, H), :], bot(my),
             sendL_sems.at[0], recvL_sems.at[0], left)

        cp_w.wait()
        gemm(x_hbm.at[pl.ds(0, H), :], my * M_PER)
        gemm(x_hbm.at[pl.ds(H, H), :], my * M_PER + H)

        for h in range(N_HOP - 1):
            o_r = lax.rem(my + N_DEV - h - 1, N_DEV)
            o_l = lax.rem(my + h + 1, N_DEV)
            wait_recv(top(o_r), recvR_sems.at[h])
            if h == 0:
                send(top(o_r), top(o_r),
                     sendR_sems.at[1], recvR_sems.at[1], right)
            else:
                for j in range(2):
                    send(top(o_r, j * Q, Q), top(o_r, j * Q, Q),
                         sendR_sems.at[2 + j], recvR_sems.at[2 + j], right)
            wait_recv(bot(o_l), recvL_sems.at[h])
            if h == 0:
                send(bot(o_l), bot(o_l),
                     sendL_sems.at[1], recvL_sems.at[1], left)
            else:
                for j in range(2):
                    send(bot(o_l, j * Q, Q), bot(o_l, j * Q, Q),
                         sendL_sems.at[2 + j], recvL_sems.at[2 + j], left)
            gemm(top(o_r), o_r * M_PER)
            gemm(bot(o_l), o_l * M_PER + H)

        o_r = lax.rem(my + 1, N_DEV)
        o_l = lax.rem(my + N_DEV - 1, N_DEV)
        for j in range(2):
            wait_recv(top(o_r, j * Q, Q), recvR_sems.at[2 + j])
            wait_recv(bot(o_l, j * Q, Q), recvL_sems.at[2 + j])
            gemm(top(o_r, j * Q, Q), o_r * M_PER + j * Q, rows=Q)
            gemm(bot(o_l, j * Q, Q), o_l * M_PER + H + j * Q, rows=Q)

        for d in send_descs:
            d.wait_send()
        for cp in pending_out:
            if cp is not None:
                cp.wait()

    out, _comm = pl.pallas_call(
        body,
        out_shape=(
            jax.ShapeDtypeStruct((N_DEV * M_PER, N_PER), jnp.float32),
            jax.ShapeDtypeStruct((N_DEV, M_PER, K), jnp.float32),
        ),
        in_specs=[
            pl.BlockSpec(memory_space=pl.ANY),
            pl.BlockSpec(memory_space=pl.ANY),
        ],
        out_specs=(
            pl.BlockSpec(memory_space=pl.ANY),
            pl.BlockSpec(memory_space=pl.ANY),
        ),
        scratch_shapes=[
            pltpu.MemorySpace.VMEM((K, N_PER), jnp.float32),
            pltpu.MemorySpace.VMEM((H, K), jnp.float32),
            pltpu.MemorySpace.VMEM((2, H, N_PER), jnp.float32),
            pltpu.SemaphoreType.DMA((N_MSG,)),
            pltpu.SemaphoreType.DMA((N_MSG,)),
            pltpu.SemaphoreType.DMA((N_MSG,)),
            pltpu.SemaphoreType.DMA((N_MSG,)),
            pltpu.SemaphoreType.DMA,
            pltpu.SemaphoreType.DMA,
            pltpu.SemaphoreType.DMA((2,)),
        ],
        compiler_params=pltpu.CompilerParams(
            collective_id=0,
            vmem_limit_bytes=64 * 1024 * 1024,
        ),
    )(x, w_mat)
    return out
